# Initial kernel scaffold; baseline (speedup 1.0000x reference)
import jax
import jax.numpy as jnp
from jax.experimental import pallas as pl


def kernel(features, rel_emb, adj, r_index, r_val, features_c, Fussion, attn_kernel, attn_kernel_ent, proxy, gate_w, gate_b):
    raise NotImplementedError("write your pallas kernel here")


# speedup vs baseline: 3.7078x; 3.7078x over previous
import functools

import jax
import jax.numpy as jnp
from jax import lax
from jax.experimental import pallas as pl
from jax.experimental.pallas import tpu as pltpu
from jax.experimental.pallas import tpu_sc as plsc

NODE = 10000
REL = 1000
TRI = 160000
DIM = 128
NC = 2
NS = 16
L = 16
AH = REL * REL // NC
AHP = 501760
ZROW = AHP // NS
RPS = 624
RTAIL = NODE - NS * RPS
NODEP = 10112

CH_A = 128
CH_B = 128

_MESH = plsc.VectorSubcoreMesh(core_axis_name="c", subcore_axis_name="s",
                               num_cores=NC, num_subcores=NS)
_CPARAMS = pltpu.CompilerParams(needs_layout_passes=False)


def _zero_1d(ref, n):
    def body(i, _):
        ref[pl.ds(i * L, L)] = jnp.zeros((L,), jnp.float32)
        return 0
    lax.fori_loop(0, n // L, body, 0)


def _zero_2d(ref, rows):
    def body(i, _):
        for k in range(DIM // L):
            ref[i, pl.ds(k * L, L)] = jnp.zeros((L,), jnp.float32)
        return 0
    lax.fori_loop(0, rows, body, 0)


def _rsqrt(x):
    i = plsc.bitcast(x, jnp.int32)
    i = jnp.int32(0x5F3759DF) - jnp.right_shift(i, jnp.int32(1))
    y = plsc.bitcast(i, jnp.float32)
    for _ in range(3):
        y = y * (jnp.float32(1.5) - jnp.float32(0.5) * x * y * y)
    return y


def _edge_pass_a(s_hbm, d_hbm, rf_hbm, rv_hbm, fc_hbm, tf_hbm,
                 acc_out, a_out, cnt_out,
                 accum, ahalf, cntsh,
                 sbuf, dbuf, rfbuf, rvbuf, aidx, aval, cidx, onesb,
                 rowbuf, zbuf):
    core = lax.axis_index("c")
    sid = lax.axis_index("s")
    is0 = core == 0
    lo = core * AH

    _zero_1d(zbuf, 1024)
    _zero_2d(rowbuf, CH_A)
    for g in range(CH_A // L):
        onesb[pl.ds(g * L, L)] = jnp.full((L,), 1.0, jnp.float32)
    base_r = sid * RPS
    for t in range(4):
        pltpu.sync_copy(rowbuf, accum.at[pl.ds(base_r + t * CH_A, CH_A), :])
    pltpu.sync_copy(rowbuf.at[pl.ds(0, RPS - 4 * CH_A)],
                    accum.at[pl.ds(base_r + 4 * CH_A, RPS - 4 * CH_A), :])

    @pl.when(sid == 0)
    def _():
        pltpu.sync_copy(rowbuf.at[pl.ds(0, RTAIL)],
                        accum.at[pl.ds(NS * RPS, RTAIL), :])
        for t in range(NODEP // 1024):
            pltpu.sync_copy(zbuf, cntsh.at[pl.ds(t * 1024, 1024)])
        pltpu.sync_copy(zbuf.at[pl.ds(0, NODEP - (NODEP // 1024) * 1024)],
                        cntsh.at[pl.ds((NODEP // 1024) * 1024,
                                       NODEP - (NODEP // 1024) * 1024)])

    zbase = sid * ZROW
    for t in range(ZROW // 1024):
        pltpu.sync_copy(zbuf, ahalf.at[pl.ds(zbase + t * 1024, 1024)])
    rem = ZROW - (ZROW // 1024) * 1024
    pltpu.sync_copy(zbuf.at[pl.ds(0, rem)],
                    ahalf.at[pl.ds(zbase + (ZROW // 1024) * 1024, rem)])

    plsc.subcore_barrier()

    n_my = (TRI // CH_A) // NS + jnp.where(sid < (TRI // CH_A) % NS, 1, 0)

    def chunk_body(k, _):
        c = k * NS + sid
        base = c * CH_A
        pltpu.sync_copy(s_hbm.at[pl.ds(base, CH_A)], sbuf)
        pltpu.sync_copy(d_hbm.at[pl.ds(base, CH_A)], dbuf)
        pltpu.sync_copy(rf_hbm.at[pl.ds(base, CH_A)], rfbuf)
        pltpu.sync_copy(rv_hbm.at[pl.ds(base, CH_A)], rvbuf)

        @pl.when(is0)
        def _():
            pltpu.sync_copy(fc_hbm.at[dbuf], rowbuf)

        @pl.when(jnp.logical_not(is0))
        def _():
            pltpu.sync_copy(tf_hbm.at[dbuf], rowbuf)

        for g in range(CH_A // L):
            sl = pl.ds(g * L, L)
            s16 = sbuf[sl]
            d16 = dbuf[sl]
            cidx[sl] = jnp.where(is0, s16, d16)
            rf = rfbuf[sl]
            rv = rvbuf[sl]
            m = jnp.logical_and(rf >= lo, rf < lo + AH)
            aidx[sl] = jnp.where(m, rf - lo, 0)
            aval[sl] = jnp.where(m, rv, jnp.float32(0.0))

        pltpu.sync_copy(rowbuf, accum.at[sbuf], add=True)
        pltpu.sync_copy(aval, ahalf.at[aidx], add=True)
        pltpu.sync_copy(onesb, cntsh.at[cidx], add=True)
        return 0

    lax.fori_loop(0, n_my, chunk_body, 0)

    plsc.subcore_barrier()

    pltpu.sync_copy(accum.at[pl.ds(base_r, RPS), :],
                    acc_out.at[core, pl.ds(base_r, RPS), :])
    pltpu.sync_copy(ahalf.at[pl.ds(zbase, ZROW)],
                    a_out.at[core, pl.ds(zbase, ZROW)])

    @pl.when(sid == 0)
    def _():
        pltpu.sync_copy(accum.at[pl.ds(NS * RPS, RTAIL), :],
                        acc_out.at[core, pl.ds(NS * RPS, RTAIL), :])
        pltpu.sync_copy(cntsh, cnt_out.at[core, :])


def _edge_pass_b(s_hbm, d_hbm, u_hbm, w_hbm, tf_hbm, ak_hbm,
                 r2_out, d2_out,
                 r2acc, d2sh,
                 sbuf, dbuf, ubuf, wbuf, fbuf, e2v, akbuf):
    core = lax.axis_index("c")
    sid = lax.axis_index("s")
    wid = core * NS + sid

    _zero_2d(fbuf, CH_B)
    _zero_1d(e2v, CH_B)
    base_r = sid * RPS
    for t in range(RPS // CH_B):
        pltpu.sync_copy(fbuf, r2acc.at[pl.ds(base_r + t * CH_B, CH_B), :])
    remr = RPS - (RPS // CH_B) * CH_B
    pltpu.sync_copy(fbuf.at[pl.ds(0, remr)],
                    r2acc.at[pl.ds(base_r + (RPS // CH_B) * CH_B, remr), :])

    @pl.when(sid == 0)
    def _():
        pltpu.sync_copy(fbuf.at[pl.ds(0, RTAIL)],
                        r2acc.at[pl.ds(NS * RPS, RTAIL), :])
        for t in range(NODEP // CH_B):
            pltpu.sync_copy(e2v, d2sh.at[pl.ds(t * CH_B, CH_B)])

    pltpu.sync_copy(ak_hbm, akbuf)
    plsc.subcore_barrier()

    akv = [akbuf[pl.ds(k * L, L)] for k in range(DIM // L)]
    lanes = lax.iota(jnp.int32, L)
    lane0 = lanes == 0

    n_my = (TRI // CH_B) // (NC * NS) + jnp.where(
        wid < (TRI // CH_B) % (NC * NS), 1, 0)

    def chunk_body(k, _):
        c = k * (NC * NS) + wid
        base = c * CH_B
        pltpu.sync_copy(s_hbm.at[pl.ds(base, CH_B)], sbuf)
        pltpu.sync_copy(d_hbm.at[pl.ds(base, CH_B)], dbuf)
        pltpu.sync_copy(u_hbm.at[sbuf], ubuf)
        pltpu.sync_copy(w_hbm.at[dbuf], wbuf)
        pltpu.sync_copy(tf_hbm.at[dbuf], fbuf)

        def edge_body(e, _):
            vs = []
            fs = []
            vv = jnp.zeros((L,), jnp.float32)
            fv = jnp.zeros((L,), jnp.float32)
            av = jnp.zeros((L,), jnp.float32)
            for kk in range(DIM // L):
                sl = pl.ds(kk * L, L)
                u = ubuf[e, sl]
                w = wbuf[e, sl]
                f = fbuf[e, sl]
                v = u + w
                vs.append(v)
                fs.append(f)
                vv = vv + v * v
                fv = fv + f * v
                av = av + akv[kk] * v
            ssv = jnp.full((L,), jnp.maximum(jnp.sum(vv), jnp.float32(1e-24)))
            fvv = jnp.full((L,), jnp.sum(fv))
            avv = jnp.full((L,), jnp.sum(av))
            rsv = _rsqrt(ssv)
            e2 = jnp.exp(avv * rsv)
            c2 = jnp.float32(2.0) * e2 * fvv / ssv
            for kk in range(DIM // L):
                fbuf[e, pl.ds(kk * L, L)] = e2 * fs[kk] - c2 * vs[kk]
            plsc.store_scatter(e2v, [jnp.full((L,), e, jnp.int32)], e2,
                               mask=lane0)
            return 0

        lax.fori_loop(0, CH_B, edge_body, 0)

        pltpu.sync_copy(fbuf, r2acc.at[sbuf], add=True)
        pltpu.sync_copy(e2v, d2sh.at[sbuf], add=True)
        return 0

    lax.fori_loop(0, n_my, chunk_body, 0)

    plsc.subcore_barrier()

    pltpu.sync_copy(r2acc.at[pl.ds(base_r, RPS), :],
                    r2_out.at[core, pl.ds(base_r, RPS), :])

    @pl.when(sid == 0)
    def _():
        pltpu.sync_copy(r2acc.at[pl.ds(NS * RPS, RTAIL), :],
                        r2_out.at[core, pl.ds(NS * RPS, RTAIL), :])
        pltpu.sync_copy(d2sh, d2_out.at[core, :])


@functools.partial(
    pl.kernel,
    out_type=(
        jax.ShapeDtypeStruct((NC, NODE, DIM), jnp.float32),
        jax.ShapeDtypeStruct((NC, AHP), jnp.float32),
        jax.ShapeDtypeStruct((NC, NODEP), jnp.float32),
    ),
    mesh=_MESH,
    compiler_params=_CPARAMS,
    scratch_types=[
        pltpu.VMEM_SHARED((NODE, DIM), jnp.float32),
        pltpu.VMEM_SHARED((AHP,), jnp.float32),
        pltpu.VMEM_SHARED((NODEP,), jnp.float32),
        pltpu.VMEM((CH_A,), jnp.int32),
        pltpu.VMEM((CH_A,), jnp.int32),
        pltpu.VMEM((CH_A,), jnp.int32),
        pltpu.VMEM((CH_A,), jnp.float32),
        pltpu.VMEM((CH_A,), jnp.int32),
        pltpu.VMEM((CH_A,), jnp.float32),
        pltpu.VMEM((CH_A,), jnp.int32),
        pltpu.VMEM((CH_A,), jnp.float32),
        pltpu.VMEM((CH_A, DIM), jnp.float32),
        pltpu.VMEM((1024,), jnp.float32),
    ],
)
def _kernel_a(*refs):
    _edge_pass_a(*refs)


@functools.partial(
    pl.kernel,
    out_type=(
        jax.ShapeDtypeStruct((NC, NODE, DIM), jnp.float32),
        jax.ShapeDtypeStruct((NC, NODEP), jnp.float32),
    ),
    mesh=_MESH,
    compiler_params=_CPARAMS,
    scratch_types=[
        pltpu.VMEM_SHARED((NODE, DIM), jnp.float32),
        pltpu.VMEM_SHARED((NODEP,), jnp.float32),
        pltpu.VMEM((CH_B,), jnp.int32),
        pltpu.VMEM((CH_B,), jnp.int32),
        pltpu.VMEM((CH_B, DIM), jnp.float32),
        pltpu.VMEM((CH_B, DIM), jnp.float32),
        pltpu.VMEM((CH_B, DIM), jnp.float32),
        pltpu.VMEM((CH_B,), jnp.float32),
        pltpu.VMEM((DIM,), jnp.float32),
    ],
)
def _kernel_b(*refs):
    _edge_pass_b(*refs)


def kernel(features, rel_emb, adj, r_index, r_val, features_c, Fussion,
           attn_kernel, attn_kernel_ent, proxy, gate_w, gate_b):
    F = features_c.astype(jnp.float32)
    tf = jnp.tanh(features.astype(jnp.float32))
    s = adj[0].astype(jnp.int32)
    d = adj[1].astype(jnp.int32)
    rflat = r_index[0].astype(jnp.int32) * REL + r_index[1].astype(jnp.int32)
    rv = r_val.astype(jnp.float32)

    acc, a_parts, cnt = _kernel_a(s, d, rflat, rv, F, tf)
    P, SumF = acc[0], acc[1]
    cs = cnt[0, :NODE]
    cd = cnt[1, :NODE]
    A = jnp.concatenate([a_parts[0, :AH], a_parts[1, :AH]]).reshape(REL, REL)

    tri = A @ rel_emb
    tn = tri / jnp.maximum(jnp.linalg.norm(tri, axis=1, keepdims=True), 1e-12)
    e1 = jnp.exp((tn @ attn_kernel)[:, 0])

    ms = (cs @ F) / TRI
    md = (cd @ F) / TRI
    Gss = (cs[:, None] * F).T @ F
    Gdd = (cd[:, None] * F).T @ F
    Gsd = F.T @ P
    m = jnp.concatenate([ms, md])
    C = jnp.block([[Gss, Gsd], [Gsd.T, Gdd]]) - TRI * jnp.outer(m, m)
    eigvals, eigvecs = jnp.linalg.eigh(C)
    idx = jnp.argsort(eigvals)[::-1][:DIM]
    V = eigvecs[:, idx]
    S = jnp.sqrt(jnp.clip(eigvals[idx], 0.0, None))
    wv = jnp.power(S + 1e-05, -0.5)
    U = F @ (V[:DIM] * wv[None, :])
    Wm = F @ (V[DIM:] * wv[None, :])

    r2_parts, d2_parts = _kernel_b(s, d, U, Wm, tf, attn_kernel_ent[:, 0])
    R2 = r2_parts.sum(axis=0)
    D2 = d2_parts[:, :NODE].sum(axis=0)

    fd0 = tf[d[:REL]]
    dt = jnp.sum(fd0 * tn, axis=1)
    nr = fd0 - 2.0 * dt[:, None] * tn
    corr = e1[:, None] * nr - fd0
    R1 = SumF.at[s[:REL]].add(corr)
    D1 = cs.at[s[:REL]].add(e1 - 1.0)

    agg = (R1 / jnp.maximum(D1, 1e-30)[:, None]
           + 0.1 * R2 / jnp.maximum(D2, 1e-30)[:, None])
    return jnp.concatenate([tf, jnp.tanh(agg)], axis=-1)
